# Initial kernel scaffold; baseline (speedup 1.0000x reference)
import jax
import jax.numpy as jnp
from jax.experimental import pallas as pl


def kernel(x, W_l1, b_l1, W_r1, W_l2, b_l2, W_r2, edge_index):
    raise NotImplementedError("write your pallas kernel here")


# speedup vs baseline: 6.8096x; 6.8096x over previous
import functools

import jax
import jax.numpy as jnp
from jax import lax
from jax.experimental import pallas as pl
from jax.experimental.pallas import tpu as pltpu
from jax.experimental.pallas import tpu_sc as plsc

N = 10000
D = 128
NC = 2
NS = 16
L = 16
NW = NC * NS
CH = 128
NB = 8
NP = 10240
RPT = NP // NS
CPR = RPT // D
CPP = 8
CW = 16


def _zero_rows(ref, nrows, width):
  z = jnp.zeros((L,), jnp.float32)

  def row(i, carry):
    for j in range(width // L):
      ref[i, pl.ds(j * L, L)] = z
    return carry

  lax.fori_loop(0, nrows, row, 0)


@functools.cache
def _sc_aggregate(n_blocks):
  mesh = plsc.VectorSubcoreMesh(core_axis_name="c", subcore_axis_name="s",
                                num_cores=NC, num_subcores=NS)
  out_type = jax.ShapeDtypeStruct((NC, NP, D), jnp.float32)

  scratch = dict(
      srcv=pltpu.VMEM((NB, CH), jnp.int32),
      dstv=pltpu.VMEM((NB, CH), jnp.int32),
      rows=pltpu.VMEM((CH, D), jnp.float32),
      sem=pltpu.SemaphoreType.DMA,
      acc_sh=pltpu.VMEM_SHARED((NP, D), jnp.float32),
  )

  def body(u_hbm, src_hbm, dst_hbm, agg_hbm, *, srcv, dstv, rows, sem,
           acc_sh):
    c = lax.axis_index("c")
    s = lax.axis_index("s")

    _zero_rows(rows, CH, D)
    base = s * RPT
    for k in range(RPT // CH):
      pltpu.sync_copy(rows, acc_sh.at[pl.ds(base + k * CH, CH)])
    plsc.subcore_barrier()

    w = c * NS + s

    def block(b, carry):
      r0 = (w * n_blocks + b) * NB
      pltpu.sync_copy(src_hbm.at[pl.ds(r0, NB)], srcv)
      pltpu.sync_copy(dst_hbm.at[pl.ds(r0, NB)], dstv)
      for k in range(NB):
        pltpu.async_copy(u_hbm.at[srcv.at[k]], rows, sem).wait()
        pltpu.sync_copy(rows, acc_sh.at[dstv.at[k]], add=True)
      return carry

    lax.fori_loop(0, n_blocks, block, 0)
    plsc.subcore_barrier()

    pltpu.sync_copy(acc_sh.at[pl.ds(base, RPT)],
                    agg_hbm.at[c, pl.ds(base, RPT)])

  return pl.kernel(body, out_type=out_type, mesh=mesh,
                   scratch_types=scratch)


@functools.cache
def _sc_count(n_blocks):
  mesh = plsc.VectorSubcoreMesh(core_axis_name="c", subcore_axis_name="s",
                                num_cores=NC, num_subcores=NS)
  out_type = jax.ShapeDtypeStruct((NC, NP, D), jnp.float32)

  scratch = dict(
      dstv=pltpu.VMEM((NB, CH), jnp.int32),
      onesv=pltpu.VMEM((CH, D), jnp.float32),
      cnt_sh=pltpu.VMEM_SHARED((NP, D), jnp.float32),
  )

  def body(dst_hbm, cnt_hbm, *, dstv, onesv, cnt_sh):
    c = lax.axis_index("c")
    s = lax.axis_index("s")

    _zero_rows(onesv, CH, D)
    base = s * RPT
    for k in range(RPT // CH):
      pltpu.sync_copy(onesv, cnt_sh.at[pl.ds(base + k * CH, CH)])
    one = jnp.ones((L,), jnp.float32)

    def ones_row(i, carry):
      for j in range(D // L):
        onesv[i, pl.ds(j * L, L)] = one
      return carry

    lax.fori_loop(0, CH, ones_row, 0)
    plsc.subcore_barrier()

    w = c * NS + s

    def block(b, carry):
      r0 = (w * n_blocks + b) * NB
      pltpu.sync_copy(dst_hbm.at[pl.ds(r0, NB)], dstv)
      for k in range(NB):
        pltpu.sync_copy(onesv, cnt_sh.at[dstv.at[k]], add=True)
      return carry

    lax.fori_loop(0, n_blocks, block, 0)
    plsc.subcore_barrier()

    pltpu.sync_copy(cnt_sh.at[pl.ds(base, RPT)],
                    cnt_hbm.at[c, pl.ds(base, RPT)])

  return pl.kernel(body, out_type=out_type, mesh=mesh,
                   scratch_types=scratch)


def _tc_layer_body(relu, agg_ref, cnt_ref, x_ref, wl_ref, wr_ref, b_ref,
                   o_ref):
  aggs = agg_ref[0] + agg_ref[1]
  cnt = cnt_ref[:, 0:1] + cnt_ref[:, 1:2]
  mean = aggs / jnp.maximum(cnt, 1.0)
  dn = (((1,), (1,)), ((), ()))
  t = (lax.dot_general(mean, wl_ref[...], dn,
                       preferred_element_type=jnp.float32)
       + lax.dot_general(x_ref[...], wr_ref[...], dn,
                         preferred_element_type=jnp.float32)
       + b_ref[...])
  o_ref[...] = jnp.maximum(t, 0.0) if relu else t


@functools.cache
def _tc_layer(relu, bm=1000):
  grid = (N // bm,)
  return pl.pallas_call(
      functools.partial(_tc_layer_body, relu),
      grid=grid,
      in_specs=[
          pl.BlockSpec((NC, bm, D), lambda i: (0, i, 0)),
          pl.BlockSpec((bm, NC), lambda i: (i, 0)),
          pl.BlockSpec((bm, D), lambda i: (i, 0)),
          pl.BlockSpec((D, D), lambda i: (0, 0)),
          pl.BlockSpec((D, D), lambda i: (0, 0)),
          pl.BlockSpec((1, D), lambda i: (0, 0)),
      ],
      out_specs=pl.BlockSpec((bm, D), lambda i: (i, 0)),
      out_shape=jax.ShapeDtypeStruct((N, D), jnp.float32),
  )


def kernel(x, W_l1, b_l1, W_r1, W_l2, b_l2, W_r2, edge_index):
  E = edge_index.shape[1]
  per_w = -(-E // (NW * CH * NB)) * CH * NB
  n_blocks = per_w // (CH * NB)
  pad = per_w * NW - E
  src = edge_index[0].astype(jnp.int32)
  dst = edge_index[1].astype(jnp.int32)
  pad_iota = jnp.arange(pad, dtype=jnp.int32)
  src = jnp.concatenate([src, pad_iota % N]).reshape(-1, CH)
  dst = jnp.concatenate([dst, N + pad_iota % (NP - N)]).reshape(-1, CH)

  cntp = _sc_count(n_blocks)(dst)
  cnt = cntp[:, :N, 0].T
  agg1 = _sc_aggregate(n_blocks)(x, src, dst)[:, :N]
  b1 = b_l1.reshape(1, D)
  h = _tc_layer(True)(agg1, cnt, x, W_l1, W_r1, b1)

  agg2 = _sc_aggregate(n_blocks)(h, src, dst)[:, :N]
  b2 = b_l2.reshape(1, D)
  return _tc_layer(False)(agg2, cnt, h, W_l2, W_r2, b2)


# speedup vs baseline: 8.6945x; 1.2768x over previous
import functools

import jax
import jax.numpy as jnp
from jax import lax
from jax.experimental import pallas as pl
from jax.experimental.pallas import tpu as pltpu
from jax.experimental.pallas import tpu_sc as plsc

N = 10000
D = 128
NC = 2
NS = 16
L = 16
NW = NC * NS
CH = 128
NB = 8
NP = 10240
RPT = NP // NS
CPR = RPT // D
CPP = 8
CW = 16


def _zero_rows(ref, nrows, width):
  z = jnp.zeros((L,), jnp.float32)

  def row(i, carry):
    for j in range(width // L):
      ref[i, pl.ds(j * L, L)] = z
    return carry

  lax.fori_loop(0, nrows, row, 0)


@functools.cache
def _sc_aggregate(n_blocks):
  mesh = plsc.VectorSubcoreMesh(core_axis_name="c", subcore_axis_name="s",
                                num_cores=NC, num_subcores=NS)
  out_type = jax.ShapeDtypeStruct((NC, NP, D), jnp.float32)

  scratch = dict(
      srcv=pltpu.VMEM((NB, CH), jnp.int32),
      dstv=pltpu.VMEM((NB, CH), jnp.int32),
      rows=pltpu.VMEM((2, CH, D), jnp.float32),
      sem0=pltpu.SemaphoreType.DMA,
      sem1=pltpu.SemaphoreType.DMA,
      acc_sh=pltpu.VMEM_SHARED((NP, D), jnp.float32),
  )

  def body(u_hbm, src_hbm, dst_hbm, agg_hbm, *, srcv, dstv, rows, sem0,
           sem1, acc_sh):
    c = lax.axis_index("c")
    s = lax.axis_index("s")
    sems = (sem0, sem1)

    _zero_rows(rows.at[0], CH, D)
    base = s * RPT
    for k in range(RPT // CH):
      pltpu.sync_copy(rows.at[0], acc_sh.at[pl.ds(base + k * CH, CH)])
    plsc.subcore_barrier()

    w = c * NS + s

    def block(b, carry):
      r0 = (w * n_blocks + b) * NB
      pltpu.sync_copy(src_hbm.at[pl.ds(r0, NB)], srcv)
      pltpu.sync_copy(dst_hbm.at[pl.ds(r0, NB)], dstv)
      descs = [None, None]
      descs[0] = pltpu.async_copy(u_hbm.at[srcv.at[0]], rows.at[0], sem0)
      for k in range(NB):
        nxt = (k + 1) % 2
        if k + 1 < NB:
          descs[nxt] = pltpu.async_copy(u_hbm.at[srcv.at[k + 1]],
                                        rows.at[nxt], sems[nxt])
        descs[k % 2].wait()
        pltpu.sync_copy(rows.at[k % 2], acc_sh.at[dstv.at[k]], add=True)
      return carry

    lax.fori_loop(0, n_blocks, block, 0)
    plsc.subcore_barrier()

    pltpu.sync_copy(acc_sh.at[pl.ds(base, RPT)],
                    agg_hbm.at[c, pl.ds(base, RPT)])

  return pl.kernel(body, out_type=out_type, mesh=mesh,
                   scratch_types=scratch)


@functools.cache
def _sc_count(n_blocks):
  mesh = plsc.VectorSubcoreMesh(core_axis_name="c", subcore_axis_name="s",
                                num_cores=NC, num_subcores=NS)
  out_type = jax.ShapeDtypeStruct((NC, NP, D), jnp.float32)

  scratch = dict(
      dstv=pltpu.VMEM((NB, CH), jnp.int32),
      onesv=pltpu.VMEM((CH, D), jnp.float32),
      sem=pltpu.SemaphoreType.DMA,
      cnt_sh=pltpu.VMEM_SHARED((NP, D), jnp.float32),
  )

  def body(dst_hbm, cnt_hbm, *, dstv, onesv, sem, cnt_sh):
    c = lax.axis_index("c")
    s = lax.axis_index("s")

    _zero_rows(onesv, CH, D)
    base = s * RPT
    for k in range(RPT // CH):
      pltpu.sync_copy(onesv, cnt_sh.at[pl.ds(base + k * CH, CH)])
    one = jnp.ones((L,), jnp.float32)

    def ones_row(i, carry):
      for j in range(D // L):
        onesv[i, pl.ds(j * L, L)] = one
      return carry

    lax.fori_loop(0, CH, ones_row, 0)
    plsc.subcore_barrier()

    w = c * NS + s

    def block(b, carry):
      r0 = (w * n_blocks + b) * NB
      pltpu.sync_copy(dst_hbm.at[pl.ds(r0, NB)], dstv)
      descs = [
          pltpu.async_copy(onesv, cnt_sh.at[dstv.at[k]], sem, add=True)
          for k in range(NB)
      ]
      for d in descs:
        d.wait()
      return carry

    lax.fori_loop(0, n_blocks, block, 0)
    plsc.subcore_barrier()

    pltpu.sync_copy(cnt_sh.at[pl.ds(base, RPT)],
                    cnt_hbm.at[c, pl.ds(base, RPT)])

  return pl.kernel(body, out_type=out_type, mesh=mesh,
                   scratch_types=scratch)


def _tc_layer_body(relu, agg_ref, cnt_ref, x_ref, wl_ref, wr_ref, b_ref,
                   o_ref):
  aggs = agg_ref[0] + agg_ref[1]
  cnt = cnt_ref[:, 0:1] + cnt_ref[:, 1:2]
  mean = aggs / jnp.maximum(cnt, 1.0)
  dn = (((1,), (1,)), ((), ()))
  t = (lax.dot_general(mean, wl_ref[...], dn,
                       preferred_element_type=jnp.float32)
       + lax.dot_general(x_ref[...], wr_ref[...], dn,
                         preferred_element_type=jnp.float32)
       + b_ref[...])
  o_ref[...] = jnp.maximum(t, 0.0) if relu else t


@functools.cache
def _tc_layer(relu, bm=1000):
  grid = (N // bm,)
  return pl.pallas_call(
      functools.partial(_tc_layer_body, relu),
      grid=grid,
      in_specs=[
          pl.BlockSpec((NC, bm, D), lambda i: (0, i, 0)),
          pl.BlockSpec((bm, NC), lambda i: (i, 0)),
          pl.BlockSpec((bm, D), lambda i: (i, 0)),
          pl.BlockSpec((D, D), lambda i: (0, 0)),
          pl.BlockSpec((D, D), lambda i: (0, 0)),
          pl.BlockSpec((1, D), lambda i: (0, 0)),
      ],
      out_specs=pl.BlockSpec((bm, D), lambda i: (i, 0)),
      out_shape=jax.ShapeDtypeStruct((N, D), jnp.float32),
  )


def kernel(x, W_l1, b_l1, W_r1, W_l2, b_l2, W_r2, edge_index):
  E = edge_index.shape[1]
  per_w = -(-E // (NW * CH * NB)) * CH * NB
  n_blocks = per_w // (CH * NB)
  pad = per_w * NW - E
  src = edge_index[0].astype(jnp.int32)
  dst = edge_index[1].astype(jnp.int32)
  pad_iota = jnp.arange(pad, dtype=jnp.int32)
  src = jnp.concatenate([src, pad_iota % N]).reshape(-1, CH)
  dst = jnp.concatenate([dst, N + pad_iota % (NP - N)]).reshape(-1, CH)

  cntp = _sc_count(n_blocks)(dst)
  cnt = cntp[:, :N, 0].T
  agg1 = _sc_aggregate(n_blocks)(x, src, dst)[:, :N]
  b1 = b_l1.reshape(1, D)
  h = _tc_layer(True)(agg1, cnt, x, W_l1, W_r1, b1)

  agg2 = _sc_aggregate(n_blocks)(h, src, dst)[:, :N]
  b2 = b_l2.reshape(1, D)
  return _tc_layer(False)(agg2, cnt, h, W_l2, W_r2, b2)


# speedup vs baseline: 9.8174x; 1.1291x over previous
import functools

import jax
import jax.numpy as jnp
from jax import lax
from jax.experimental import pallas as pl
from jax.experimental.pallas import tpu as pltpu
from jax.experimental.pallas import tpu_sc as plsc

N = 10000
D = 128
NC = 2
NS = 16
L = 16
NW = NC * NS
CH = 128
NB = 8
NP = 10240
RPT = NP // NS
CPR = RPT // D
CPP = 8
CW = 16


def _zero_rows(ref, nrows, width):
  z = jnp.zeros((L,), jnp.float32)

  def row(i, carry):
    for j in range(width // L):
      ref[i, pl.ds(j * L, L)] = z
    return carry

  lax.fori_loop(0, nrows, row, 0)


@functools.cache
def _sc_aggregate(n_blocks):
  mesh = plsc.VectorSubcoreMesh(core_axis_name="c", subcore_axis_name="s",
                                num_cores=NC, num_subcores=NS)
  out_type = jax.ShapeDtypeStruct((NC, NP, D), jnp.float32)

  scratch = dict(
      srcv=pltpu.VMEM((2, NB, CH), jnp.int32),
      dstv=pltpu.VMEM((2, NB, CH), jnp.int32),
      rows=pltpu.VMEM((2, CH, D), jnp.float32),
      sem0=pltpu.SemaphoreType.DMA,
      sem1=pltpu.SemaphoreType.DMA,
      semi=pltpu.SemaphoreType.DMA,
      acc_sh=pltpu.VMEM_SHARED((NP, D), jnp.float32),
  )

  def body(u_hbm, src_hbm, dst_hbm, agg_hbm, *, srcv, dstv, rows, sem0,
           sem1, semi, acc_sh):
    c = lax.axis_index("c")
    s = lax.axis_index("s")
    gsems = (sem0, sem1)

    _zero_rows(rows.at[0], CH, D)
    base = s * RPT
    for k in range(RPT // CH):
      pltpu.sync_copy(rows.at[0], acc_sh.at[pl.ds(base + k * CH, CH)])
    plsc.subcore_barrier()

    w = c * NS + s

    def r0(b):
      return (w * n_blocks + b) * NB

    pltpu.sync_copy(src_hbm.at[pl.ds(r0(0), NB)], srcv.at[0])
    pltpu.sync_copy(dst_hbm.at[pl.ds(r0(0), NB)], dstv.at[0])
    total = n_blocks * NB
    gd = [None, None]
    idxd = [None, None]
    gd[0] = pltpu.async_copy(u_hbm.at[srcv.at[0, 0]], rows.at[0], sem0)
    for g in range(total):
      b, k = divmod(g, NB)
      ib = b % 2
      if k == 0 and b + 1 < n_blocks:
        nb_ = (b + 1) % 2
        idxd[nb_] = (
            pltpu.async_copy(src_hbm.at[pl.ds(r0(b + 1), NB)],
                             srcv.at[nb_], semi),
            pltpu.async_copy(dst_hbm.at[pl.ds(r0(b + 1), NB)],
                             dstv.at[nb_], semi),
        )
      if g + 1 < total:
        b2, k2 = divmod(g + 1, NB)
        ib2 = b2 % 2
        if k2 == 0:
          for d in idxd[ib2]:
            d.wait()
        gbuf = (g + 1) % 2
        gd[gbuf] = pltpu.async_copy(u_hbm.at[srcv.at[ib2, k2]],
                                    rows.at[gbuf], gsems[gbuf])
      gd[g % 2].wait()
      pltpu.sync_copy(rows.at[g % 2], acc_sh.at[dstv.at[ib, k]], add=True)
    plsc.subcore_barrier()

    pltpu.sync_copy(acc_sh.at[pl.ds(base, RPT)],
                    agg_hbm.at[c, pl.ds(base, RPT)])

  return pl.kernel(body, out_type=out_type, mesh=mesh,
                   scratch_types=scratch)


@functools.cache
def _sc_count(n_blocks):
  mesh = plsc.VectorSubcoreMesh(core_axis_name="c", subcore_axis_name="s",
                                num_cores=NC, num_subcores=NS)
  out_type = jax.ShapeDtypeStruct((NC, NP, D), jnp.float32)

  scratch = dict(
      dstv=pltpu.VMEM((2, NB, CH), jnp.int32),
      onesv=pltpu.VMEM((CH, D), jnp.float32),
      sem=pltpu.SemaphoreType.DMA,
      semi=pltpu.SemaphoreType.DMA,
      cnt_sh=pltpu.VMEM_SHARED((NP, D), jnp.float32),
  )

  def body(dst_hbm, cnt_hbm, *, dstv, onesv, sem, semi, cnt_sh):
    c = lax.axis_index("c")
    s = lax.axis_index("s")

    _zero_rows(onesv, CH, D)
    base = s * RPT
    for k in range(RPT // CH):
      pltpu.sync_copy(onesv, cnt_sh.at[pl.ds(base + k * CH, CH)])
    one = jnp.ones((L,), jnp.float32)

    def ones_row(i, carry):
      for j in range(D // L):
        onesv[i, pl.ds(j * L, L)] = one
      return carry

    lax.fori_loop(0, CH, ones_row, 0)
    plsc.subcore_barrier()

    w = c * NS + s

    def r0(b):
      return (w * n_blocks + b) * NB

    pltpu.sync_copy(dst_hbm.at[pl.ds(r0(0), NB)], dstv.at[0])
    prev = []
    idxd = None
    for b in range(n_blocks):
      ib = b % 2
      for d in prev:
        d.wait()
      if b + 1 < n_blocks:
        idxd = pltpu.async_copy(dst_hbm.at[pl.ds(r0(b + 1), NB)],
                                dstv.at[(b + 1) % 2], semi)
      prev = [
          pltpu.async_copy(onesv, cnt_sh.at[dstv.at[ib, k]], sem,
                           add=True)
          for k in range(NB)
      ]
      if b + 1 < n_blocks:
        idxd.wait()
    for d in prev:
      d.wait()
    plsc.subcore_barrier()

    pltpu.sync_copy(cnt_sh.at[pl.ds(base, RPT)],
                    cnt_hbm.at[c, pl.ds(base, RPT)])

  return pl.kernel(body, out_type=out_type, mesh=mesh,
                   scratch_types=scratch)


def _tc_layer_body(relu, agg_ref, cnt_ref, x_ref, wl_ref, wr_ref, b_ref,
                   o_ref):
  aggs = agg_ref[0] + agg_ref[1]
  cnt = cnt_ref[:, 0:1] + cnt_ref[:, 1:2]
  mean = aggs / jnp.maximum(cnt, 1.0)
  dn = (((1,), (1,)), ((), ()))
  t = (lax.dot_general(mean, wl_ref[...], dn,
                       preferred_element_type=jnp.float32)
       + lax.dot_general(x_ref[...], wr_ref[...], dn,
                         preferred_element_type=jnp.float32)
       + b_ref[...])
  o_ref[...] = jnp.maximum(t, 0.0) if relu else t


@functools.cache
def _tc_layer(relu, bm=1000):
  grid = (N // bm,)
  return pl.pallas_call(
      functools.partial(_tc_layer_body, relu),
      grid=grid,
      in_specs=[
          pl.BlockSpec((NC, bm, D), lambda i: (0, i, 0)),
          pl.BlockSpec((bm, NC), lambda i: (i, 0)),
          pl.BlockSpec((bm, D), lambda i: (i, 0)),
          pl.BlockSpec((D, D), lambda i: (0, 0)),
          pl.BlockSpec((D, D), lambda i: (0, 0)),
          pl.BlockSpec((1, D), lambda i: (0, 0)),
      ],
      out_specs=pl.BlockSpec((bm, D), lambda i: (i, 0)),
      out_shape=jax.ShapeDtypeStruct((N, D), jnp.float32),
  )


def kernel(x, W_l1, b_l1, W_r1, W_l2, b_l2, W_r2, edge_index):
  E = edge_index.shape[1]
  per_w = -(-E // (NW * CH * NB)) * CH * NB
  n_blocks = per_w // (CH * NB)
  pad = per_w * NW - E
  src = edge_index[0].astype(jnp.int32)
  dst = edge_index[1].astype(jnp.int32)
  pad_iota = jnp.arange(pad, dtype=jnp.int32)
  src = jnp.concatenate([src, pad_iota % N]).reshape(-1, CH)
  dst = jnp.concatenate([dst, N + pad_iota % (NP - N)]).reshape(-1, CH)

  cntp = _sc_count(n_blocks)(dst)
  cnt = cntp[:, :N, 0].T
  agg1 = _sc_aggregate(n_blocks)(x, src, dst)[:, :N]
  b1 = b_l1.reshape(1, D)
  h = _tc_layer(True)(agg1, cnt, x, W_l1, W_r1, b1)

  agg2 = _sc_aggregate(n_blocks)(h, src, dst)[:, :N]
  b2 = b_l2.reshape(1, D)
  return _tc_layer(False)(agg2, cnt, h, W_l2, W_r2, b2)


# speedup vs baseline: 10.3233x; 1.0515x over previous
import functools

import jax
import jax.numpy as jnp
from jax import lax
from jax.experimental import pallas as pl
from jax.experimental.pallas import tpu as pltpu
from jax.experimental.pallas import tpu_sc as plsc

N = 10000
D = 128
NC = 2
NS = 16
L = 16
NW = NC * NS
CH = 128
NB = 8
NP = 10240
RPT = NP // NS


def _zero_rows(ref, nrows, width):
  z = jnp.zeros((L,), jnp.float32)

  def row(i, carry):
    for j in range(width // L):
      ref[i, pl.ds(j * L, L)] = z
    return carry

  lax.fori_loop(0, nrows, row, 0)


@functools.cache
def _sc_aggregate(n_blocks):
  mesh = plsc.VectorSubcoreMesh(core_axis_name="c", subcore_axis_name="s",
                                num_cores=NC, num_subcores=NS)
  out_type = jax.ShapeDtypeStruct((NC, NP, D), jnp.float32)

  scratch = dict(
      srcv=pltpu.VMEM((2, NB, CH), jnp.int32),
      dstv=pltpu.VMEM((2, NB, CH), jnp.int32),
      rows=pltpu.VMEM((2, CH, D), jnp.float32),
      sem0=pltpu.SemaphoreType.DMA,
      sem1=pltpu.SemaphoreType.DMA,
      semi=pltpu.SemaphoreType.DMA,
      acc_sh=pltpu.VMEM_SHARED((NP, D), jnp.float32),
  )

  def body(u_hbm, src_hbm, dst_hbm, agg_hbm, *, srcv, dstv, rows, sem0,
           sem1, semi, acc_sh):
    c = lax.axis_index("c")
    s = lax.axis_index("s")
    gsems = (sem0, sem1)

    _zero_rows(rows.at[0], CH, D)
    base = s * RPT
    for d in [pltpu.async_copy(rows.at[0],
                               acc_sh.at[pl.ds(base + k * CH, CH)], semi)
              for k in range(RPT // CH)]:
      d.wait()
    plsc.subcore_barrier()

    w = c * NS + s

    def r0(b):
      return (w * n_blocks + b) * NB

    pltpu.sync_copy(src_hbm.at[pl.ds(r0(0), NB)], srcv.at[0])
    pltpu.sync_copy(dst_hbm.at[pl.ds(r0(0), NB)], dstv.at[0])
    total = n_blocks * NB
    gd = [None, None]
    idxd = [None, None]
    gd[0] = pltpu.async_copy(u_hbm.at[srcv.at[0, 0]], rows.at[0], sem0)
    for g in range(total):
      b, k = divmod(g, NB)
      ib = b % 2
      if k == 0 and b + 1 < n_blocks:
        nb_ = (b + 1) % 2
        idxd[nb_] = (
            pltpu.async_copy(src_hbm.at[pl.ds(r0(b + 1), NB)],
                             srcv.at[nb_], semi),
            pltpu.async_copy(dst_hbm.at[pl.ds(r0(b + 1), NB)],
                             dstv.at[nb_], semi),
        )
      if g + 1 < total:
        b2, k2 = divmod(g + 1, NB)
        ib2 = b2 % 2
        if k2 == 0:
          for d in idxd[ib2]:
            d.wait()
        gbuf = (g + 1) % 2
        gd[gbuf] = pltpu.async_copy(u_hbm.at[srcv.at[ib2, k2]],
                                    rows.at[gbuf], gsems[gbuf])
      gd[g % 2].wait()
      pltpu.sync_copy(rows.at[g % 2], acc_sh.at[dstv.at[ib, k]], add=True)
    plsc.subcore_barrier()

    pltpu.sync_copy(acc_sh.at[pl.ds(base, RPT)],
                    agg_hbm.at[c, pl.ds(base, RPT)])

  return pl.kernel(body, out_type=out_type, mesh=mesh,
                   scratch_types=scratch)


@functools.cache
def _sc_count(n_blocks):
  mesh = plsc.VectorSubcoreMesh(core_axis_name="c", subcore_axis_name="s",
                                num_cores=NC, num_subcores=NS)
  out_type = jax.ShapeDtypeStruct((NC, NP, D), jnp.float32)

  scratch = dict(
      dstv=pltpu.VMEM((2, NB, CH), jnp.int32),
      onesv=pltpu.VMEM((CH, D), jnp.float32),
      sem=pltpu.SemaphoreType.DMA,
      semi=pltpu.SemaphoreType.DMA,
      cnt_sh=pltpu.VMEM_SHARED((NP, D), jnp.float32),
  )

  def body(dst_hbm, cnt_hbm, *, dstv, onesv, sem, semi, cnt_sh):
    c = lax.axis_index("c")
    s = lax.axis_index("s")

    _zero_rows(onesv, CH, D)
    base = s * RPT
    zds = [pltpu.async_copy(onesv, cnt_sh.at[pl.ds(base + k * CH, CH)],
                            semi)
           for k in range(RPT // CH)]
    one = jnp.ones((L,), jnp.float32)

    def ones_row(i, carry):
      for j in range(D // L):
        onesv[i, pl.ds(j * L, L)] = one
      return carry

    for d in zds:
      d.wait()
    lax.fori_loop(0, CH, ones_row, 0)
    plsc.subcore_barrier()

    w = c * NS + s

    def r0(b):
      return (w * n_blocks + b) * NB

    pltpu.sync_copy(dst_hbm.at[pl.ds(r0(0), NB)], dstv.at[0])
    prev = []
    idxd = None
    for b in range(n_blocks):
      ib = b % 2
      for d in prev:
        d.wait()
      if b + 1 < n_blocks:
        idxd = pltpu.async_copy(dst_hbm.at[pl.ds(r0(b + 1), NB)],
                                dstv.at[(b + 1) % 2], semi)
      prev = [
          pltpu.async_copy(onesv, cnt_sh.at[dstv.at[ib, k]], sem,
                           add=True)
          for k in range(NB)
      ]
      if b + 1 < n_blocks:
        idxd.wait()
    for d in prev:
      d.wait()
    plsc.subcore_barrier()

    pltpu.sync_copy(cnt_sh.at[pl.ds(base, RPT)],
                    cnt_hbm.at[c, pl.ds(base, RPT)])

  return pl.kernel(body, out_type=out_type, mesh=mesh,
                   scratch_types=scratch)


def _tc_layer_body(relu, agg_ref, cnt_ref, x_ref, wl_ref, wr_ref, b_ref,
                   o_ref):
  aggs = agg_ref[0] + agg_ref[1]
  cnt = cnt_ref[0, :, 0:1] + cnt_ref[1, :, 0:1]
  mean = aggs / jnp.maximum(cnt, 1.0)
  dn = (((1,), (1,)), ((), ()))
  t = (lax.dot_general(mean, wl_ref[...], dn,
                       preferred_element_type=jnp.float32)
       + lax.dot_general(x_ref[...], wr_ref[...], dn,
                         preferred_element_type=jnp.float32)
       + b_ref[...])
  o_ref[...] = jnp.maximum(t, 0.0) if relu else t


@functools.cache
def _tc_layer(relu, bm=1000):
  grid = (N // bm,)
  return pl.pallas_call(
      functools.partial(_tc_layer_body, relu),
      grid=grid,
      in_specs=[
          pl.BlockSpec((NC, bm, D), lambda i: (0, i, 0)),
          pl.BlockSpec((NC, bm, D), lambda i: (0, i, 0)),
          pl.BlockSpec((bm, D), lambda i: (i, 0)),
          pl.BlockSpec((D, D), lambda i: (0, 0)),
          pl.BlockSpec((D, D), lambda i: (0, 0)),
          pl.BlockSpec((1, D), lambda i: (0, 0)),
      ],
      out_specs=pl.BlockSpec((bm, D), lambda i: (i, 0)),
      out_shape=jax.ShapeDtypeStruct((N, D), jnp.float32),
  )


def kernel(x, W_l1, b_l1, W_r1, W_l2, b_l2, W_r2, edge_index):
  E = edge_index.shape[1]
  per_w = -(-E // (NW * CH * NB)) * CH * NB
  n_blocks = per_w // (CH * NB)
  pad = per_w * NW - E
  src = edge_index[0].astype(jnp.int32)
  dst = edge_index[1].astype(jnp.int32)
  pad_iota = jnp.arange(pad, dtype=jnp.int32)
  src = jnp.concatenate([src, pad_iota % N]).reshape(-1, CH)
  dst = jnp.concatenate([dst, N + pad_iota % (NP - N)]).reshape(-1, CH)

  cntp = _sc_count(n_blocks)(dst)
  agg1 = _sc_aggregate(n_blocks)(x, src, dst)
  b1 = b_l1.reshape(1, D)
  h = _tc_layer(True)(agg1, cntp, x, W_l1, W_r1, b1)

  agg2 = _sc_aggregate(n_blocks)(h, src, dst)
  b2 = b_l2.reshape(1, D)
  return _tc_layer(False)(agg2, cntp, h, W_l2, W_r2, b2)
